# baseline (device time: 78829 ns/iter reference)
import jax
import jax.numpy as jnp
from jax import lax
from jax.experimental import pallas as pl
from jax.experimental.pallas import tpu as pltpu

N_DEV = 16


def kernel(x, Wg, Wu, Wd):
    m, k = x.shape
    h = Wg.shape[1]
    d = Wd.shape[1]

    def body(x_ref, wg_ref, wu_ref, wd_ref, out_ref, comm_ref, send_sems, recv_sems):
        my = lax.axis_index("i")
        left = lax.rem(my - 1 + N_DEV, N_DEV)
        right = lax.rem(my + 1, N_DEV)

        barrier_sem = pltpu.get_barrier_semaphore()
        for nbr in [left, right]:
            pl.semaphore_signal(
                barrier_sem, inc=1,
                device_id=(nbr,), device_id_type=pl.DeviceIdType.MESH,
            )
        pl.semaphore_wait(barrier_sem, 2)

        xv = x_ref[...]
        gate = jnp.dot(xv, wg_ref[...], preferred_element_type=jnp.float32)
        up = jnp.dot(xv, wu_ref[...], preferred_element_type=jnp.float32)
        act = gate * (up * jax.nn.sigmoid(up))
        partial = jnp.dot(act, wd_ref[...], preferred_element_type=jnp.float32)

        comm_ref[0] = partial
        out_ref[...] = partial

        for t in range(N_DEV - 1):
            rdma = pltpu.make_async_remote_copy(
                src_ref=comm_ref.at[t],
                dst_ref=comm_ref.at[t + 1],
                send_sem=send_sems.at[t],
                recv_sem=recv_sems.at[t],
                device_id=(right,),
                device_id_type=pl.DeviceIdType.MESH,
            )
            rdma.start()
            rdma.wait()
            out_ref[...] += comm_ref[t + 1]

    return pl.pallas_call(
        body,
        out_shape=jax.ShapeDtypeStruct((m, d), jnp.float32),
        in_specs=[
            pl.BlockSpec(memory_space=pltpu.VMEM),
            pl.BlockSpec(memory_space=pltpu.VMEM),
            pl.BlockSpec(memory_space=pltpu.VMEM),
            pl.BlockSpec(memory_space=pltpu.VMEM),
        ],
        out_specs=pl.BlockSpec(memory_space=pltpu.VMEM),
        scratch_shapes=[
            pltpu.VMEM((N_DEV, m, d), jnp.float32),
            pltpu.SemaphoreType.DMA((N_DEV - 1,)),
            pltpu.SemaphoreType.DMA((N_DEV - 1,)),
        ],
        compiler_params=pltpu.CompilerParams(collective_id=0),
    )(x, Wg, Wu, Wd)


# device time: 21698 ns/iter; 3.6330x vs baseline; 3.6330x over previous
import jax
import jax.numpy as jnp
from jax import lax
from jax.experimental import pallas as pl
from jax.experimental.pallas import tpu as pltpu

N_DEV = 16


def kernel(x, Wg, Wu, Wd):
    m, k = x.shape
    h = Wg.shape[1]
    d = Wd.shape[1]
    rows = m // N_DEV

    def body(x_ref, wg_ref, wu_ref, wd_ref, out_ref,
             part_ref, rs_buf,
             send_sems_rs, recv_sems_rs, send_sems_ag, recv_sems_ag):
        my = lax.axis_index("i")

        barrier_sem = pltpu.get_barrier_semaphore()
        for off in range(1, N_DEV):
            pl.semaphore_signal(
                barrier_sem, inc=1,
                device_id=(lax.rem(my + off, N_DEV),),
                device_id_type=pl.DeviceIdType.MESH,
            )
        pl.semaphore_wait(barrier_sem, N_DEV - 1)

        xv = x_ref[...]
        gate = jnp.dot(xv, wg_ref[...], preferred_element_type=jnp.float32)
        up = jnp.dot(xv, wu_ref[...], preferred_element_type=jnp.float32)
        act = gate * (up * jax.nn.sigmoid(up))
        part_ref[...] = jnp.dot(act, wd_ref[...], preferred_element_type=jnp.float32)

        rs_sends = []
        for off in range(1, N_DEV):
            j = lax.rem(my + off, N_DEV)
            rdma = pltpu.make_async_remote_copy(
                src_ref=part_ref.at[pl.ds(j * rows, rows), :],
                dst_ref=rs_buf.at[my],
                send_sem=send_sems_rs.at[j],
                recv_sem=recv_sems_rs.at[my],
                device_id=(j,),
                device_id_type=pl.DeviceIdType.MESH,
            )
            rdma.start()
            rs_sends.append(rdma)
        rs_buf[my] = part_ref[pl.ds(my * rows, rows), :]

        for off in range(1, N_DEV):
            s = lax.rem(my + off, N_DEV)
            recv = pltpu.make_async_remote_copy(
                src_ref=part_ref.at[pl.ds(s * rows, rows), :],
                dst_ref=rs_buf.at[s],
                send_sem=send_sems_rs.at[s],
                recv_sem=recv_sems_rs.at[s],
                device_id=(s,),
                device_id_type=pl.DeviceIdType.MESH,
            )
            recv.wait_recv()

        reduced = jnp.sum(rs_buf[...], axis=0)
        out_ref[pl.ds(my * rows, rows), :] = reduced

        ag_sends = []
        for off in range(1, N_DEV):
            j = lax.rem(my + off, N_DEV)
            rdma = pltpu.make_async_remote_copy(
                src_ref=out_ref.at[pl.ds(my * rows, rows), :],
                dst_ref=out_ref.at[pl.ds(my * rows, rows), :],
                send_sem=send_sems_ag.at[j],
                recv_sem=recv_sems_ag.at[my],
                device_id=(j,),
                device_id_type=pl.DeviceIdType.MESH,
            )
            rdma.start()
            ag_sends.append(rdma)

        for off in range(1, N_DEV):
            s = lax.rem(my + off, N_DEV)
            recv = pltpu.make_async_remote_copy(
                src_ref=out_ref.at[pl.ds(s * rows, rows), :],
                dst_ref=out_ref.at[pl.ds(s * rows, rows), :],
                send_sem=send_sems_ag.at[s],
                recv_sem=recv_sems_ag.at[s],
                device_id=(s,),
                device_id_type=pl.DeviceIdType.MESH,
            )
            recv.wait_recv()

        for rdma in rs_sends:
            rdma.wait_send()
        for rdma in ag_sends:
            rdma.wait_send()

    return pl.pallas_call(
        body,
        out_shape=jax.ShapeDtypeStruct((m, d), jnp.float32),
        in_specs=[
            pl.BlockSpec(memory_space=pltpu.VMEM),
            pl.BlockSpec(memory_space=pltpu.VMEM),
            pl.BlockSpec(memory_space=pltpu.VMEM),
            pl.BlockSpec(memory_space=pltpu.VMEM),
        ],
        out_specs=pl.BlockSpec(memory_space=pltpu.VMEM),
        scratch_shapes=[
            pltpu.VMEM((m, d), jnp.float32),
            pltpu.VMEM((N_DEV, m // N_DEV, d), jnp.float32),
            pltpu.SemaphoreType.DMA((N_DEV,)),
            pltpu.SemaphoreType.DMA((N_DEV,)),
            pltpu.SemaphoreType.DMA((N_DEV,)),
            pltpu.SemaphoreType.DMA((N_DEV,)),
        ],
        compiler_params=pltpu.CompilerParams(collective_id=0),
    )(x, Wg, Wu, Wd)


# device time: 21126 ns/iter; 3.7314x vs baseline; 1.0271x over previous
import jax
import jax.numpy as jnp
from jax import lax
from jax.experimental import pallas as pl
from jax.experimental.pallas import tpu as pltpu

N_DEV = 16


def kernel(x, Wg, Wu, Wd):
    m, k = x.shape
    h = Wg.shape[1]
    d = Wd.shape[1]
    rows = m // N_DEV

    def body(x_ref, wg_ref, wu_ref, wd_ref, out_ref,
             part_ref, rs_buf,
             send_sems_rs, recv_sems_rs, send_sems_ag, recv_sems_ag):
        my = lax.axis_index("i")

        barrier_sem = pltpu.get_barrier_semaphore()
        for off in range(1, N_DEV):
            pl.semaphore_signal(
                barrier_sem, inc=1,
                device_id=(lax.rem(my + off, N_DEV),),
                device_id_type=pl.DeviceIdType.MESH,
            )

        xv = x_ref[...]
        gate = jnp.dot(xv, wg_ref[...], preferred_element_type=jnp.float32)
        up = jnp.dot(xv, wu_ref[...], preferred_element_type=jnp.float32)
        act = gate * (up * jax.nn.sigmoid(up))
        part_ref[...] = jnp.dot(act, wd_ref[...], preferred_element_type=jnp.float32)

        pl.semaphore_wait(barrier_sem, N_DEV - 1)

        rs_sends = []
        for off in range(1, N_DEV):
            j = lax.rem(my + off, N_DEV)
            rdma = pltpu.make_async_remote_copy(
                src_ref=part_ref.at[pl.ds(j * rows, rows), :],
                dst_ref=rs_buf.at[my],
                send_sem=send_sems_rs.at[j],
                recv_sem=recv_sems_rs.at[my],
                device_id=(j,),
                device_id_type=pl.DeviceIdType.MESH,
            )
            rdma.start()
            rs_sends.append(rdma)
        rs_buf[my] = part_ref[pl.ds(my * rows, rows), :]

        for off in range(1, N_DEV):
            s = lax.rem(my + off, N_DEV)
            recv = pltpu.make_async_remote_copy(
                src_ref=part_ref.at[pl.ds(s * rows, rows), :],
                dst_ref=rs_buf.at[s],
                send_sem=send_sems_rs.at[s],
                recv_sem=recv_sems_rs.at[s],
                device_id=(s,),
                device_id_type=pl.DeviceIdType.MESH,
            )
            recv.wait_recv()

        reduced = jnp.sum(rs_buf[...], axis=0)
        out_ref[pl.ds(my * rows, rows), :] = reduced

        ag_sends = []
        for off in range(1, N_DEV):
            j = lax.rem(my + off, N_DEV)
            rdma = pltpu.make_async_remote_copy(
                src_ref=out_ref.at[pl.ds(my * rows, rows), :],
                dst_ref=out_ref.at[pl.ds(my * rows, rows), :],
                send_sem=send_sems_ag.at[j],
                recv_sem=recv_sems_ag.at[my],
                device_id=(j,),
                device_id_type=pl.DeviceIdType.MESH,
            )
            rdma.start()
            ag_sends.append(rdma)

        for off in range(1, N_DEV):
            s = lax.rem(my + off, N_DEV)
            recv = pltpu.make_async_remote_copy(
                src_ref=out_ref.at[pl.ds(s * rows, rows), :],
                dst_ref=out_ref.at[pl.ds(s * rows, rows), :],
                send_sem=send_sems_ag.at[s],
                recv_sem=recv_sems_ag.at[s],
                device_id=(s,),
                device_id_type=pl.DeviceIdType.MESH,
            )
            recv.wait_recv()

        for rdma in rs_sends:
            rdma.wait_send()
        for rdma in ag_sends:
            rdma.wait_send()

    return pl.pallas_call(
        body,
        out_shape=jax.ShapeDtypeStruct((m, d), jnp.float32),
        in_specs=[
            pl.BlockSpec(memory_space=pltpu.VMEM),
            pl.BlockSpec(memory_space=pltpu.VMEM),
            pl.BlockSpec(memory_space=pltpu.VMEM),
            pl.BlockSpec(memory_space=pltpu.VMEM),
        ],
        out_specs=pl.BlockSpec(memory_space=pltpu.VMEM),
        scratch_shapes=[
            pltpu.VMEM((m, d), jnp.float32),
            pltpu.VMEM((N_DEV, m // N_DEV, d), jnp.float32),
            pltpu.SemaphoreType.DMA((N_DEV,)),
            pltpu.SemaphoreType.DMA((N_DEV,)),
            pltpu.SemaphoreType.DMA((N_DEV,)),
            pltpu.SemaphoreType.DMA((N_DEV,)),
        ],
        compiler_params=pltpu.CompilerParams(collective_id=0),
    )(x, Wg, Wu, Wd)


# device time: 17939 ns/iter; 4.3943x vs baseline; 1.1777x over previous
import jax
import jax.numpy as jnp
from jax import lax
from jax.experimental import pallas as pl
from jax.experimental.pallas import tpu as pltpu

N_DEV = 16


def kernel(x, Wg, Wu, Wd):
    m, k = x.shape
    h = Wg.shape[1]
    d = Wd.shape[1]
    rows = m // N_DEV

    def body(x_ref, wg_ref, wu_ref, wd_ref, out_ref,
             part_ref, rs_buf,
             send_sems_rs, recv_sems_rs, send_sems_ag, recv_sems_ag):
        my = lax.axis_index("i")

        barrier_sem = pltpu.get_barrier_semaphore()
        for off in range(1, N_DEV):
            pl.semaphore_signal(
                barrier_sem, inc=1,
                device_id=(lax.rem(my + off, N_DEV),),
                device_id_type=pl.DeviceIdType.MESH,
            )

        xv = x_ref[...]
        gate = jnp.dot(xv, wg_ref[...], preferred_element_type=jnp.float32)
        up = jnp.dot(xv, wu_ref[...], preferred_element_type=jnp.float32)
        act = gate * (up * jax.nn.sigmoid(up))
        part_ref[...] = jnp.dot(act, wd_ref[...], preferred_element_type=jnp.float32)

        pl.semaphore_wait(barrier_sem, N_DEV - 1)

        rs_sends = []
        for off in range(1, N_DEV):
            j = lax.rem(my + off, N_DEV)
            rdma = pltpu.make_async_remote_copy(
                src_ref=part_ref.at[pl.ds(j * rows, rows), :],
                dst_ref=rs_buf.at[my],
                send_sem=send_sems_rs.at[j],
                recv_sem=recv_sems_rs.at[my],
                device_id=(j,),
                device_id_type=pl.DeviceIdType.MESH,
            )
            rdma.start()
            rs_sends.append(rdma)
        rs_buf[my] = part_ref[pl.ds(my * rows, rows), :]

        for off in range(1, N_DEV):
            s = lax.rem(my + off, N_DEV)
            recv = pltpu.make_async_remote_copy(
                src_ref=part_ref.at[pl.ds(s * rows, rows), :],
                dst_ref=rs_buf.at[s],
                send_sem=send_sems_rs.at[s],
                recv_sem=recv_sems_rs.at[s],
                device_id=(s,),
                device_id_type=pl.DeviceIdType.MESH,
            )
            recv.wait_recv()

        reduced = jnp.sum(rs_buf[...], axis=0)
        out_ref[pl.ds(my * rows, rows), :] = reduced

        ag_sends = []
        for off in range(1, N_DEV):
            j = lax.rem(my + off, N_DEV)
            rdma = pltpu.make_async_remote_copy(
                src_ref=out_ref.at[pl.ds(my * rows, rows), :],
                dst_ref=out_ref.at[pl.ds(my * rows, rows), :],
                send_sem=send_sems_ag.at[j],
                recv_sem=recv_sems_ag.at[my],
                device_id=(j,),
                device_id_type=pl.DeviceIdType.MESH,
            )
            rdma.start()
            ag_sends.append(rdma)

        for off in range(1, N_DEV):
            s = lax.rem(my + off, N_DEV)
            recv = pltpu.make_async_remote_copy(
                src_ref=out_ref.at[pl.ds(s * rows, rows), :],
                dst_ref=out_ref.at[pl.ds(s * rows, rows), :],
                send_sem=send_sems_ag.at[s],
                recv_sem=recv_sems_ag.at[s],
                device_id=(s,),
                device_id_type=pl.DeviceIdType.MESH,
            )
            recv.wait_recv()

        for rdma in rs_sends:
            rdma.wait_send()
        for rdma in ag_sends:
            rdma.wait_send()

        for off in range(1, N_DEV):
            pl.semaphore_signal(
                barrier_sem, inc=1,
                device_id=(lax.rem(my + off, N_DEV),),
                device_id_type=pl.DeviceIdType.MESH,
            )

    return pl.pallas_call(
        body,
        out_shape=jax.ShapeDtypeStruct((m, d), jnp.float32),
        in_specs=[
            pl.BlockSpec(memory_space=pltpu.VMEM),
            pl.BlockSpec(memory_space=pltpu.VMEM),
            pl.BlockSpec(memory_space=pltpu.VMEM),
            pl.BlockSpec(memory_space=pltpu.VMEM),
        ],
        out_specs=pl.BlockSpec(memory_space=pltpu.VMEM),
        scratch_shapes=[
            pltpu.VMEM((m, d), jnp.float32),
            pltpu.VMEM((N_DEV, m // N_DEV, d), jnp.float32),
            pltpu.SemaphoreType.DMA((N_DEV,)),
            pltpu.SemaphoreType.DMA((N_DEV,)),
            pltpu.SemaphoreType.DMA((N_DEV,)),
            pltpu.SemaphoreType.DMA((N_DEV,)),
        ],
        compiler_params=pltpu.CompilerParams(collective_id=0),
    )(x, Wg, Wu, Wd)
